# baseline (speedup 1.0000x reference)
import functools

import jax
import jax.numpy as jnp
from jax import lax
from jax.experimental import pallas as pl
from jax.experimental.pallas import tpu as pltpu
from jax.experimental.pallas import tpu_sc as plsc

_N, _E, _D, _DE, _H = 10000, 320000, 128, 16, 16

_NC, _NS = 2, 16
_NW = _NC * _NS
_CH = 1280
_NCHUNK = _E // _CH
_TMAX = -(-_NCHUNK // _NW)
_SUB = 128
_NSUB = _CH // _SUB
_CROWS = _CH // 8


_BN = 2000


def _ab_body(x_ref, wa_ref, wb_ref, a_ref, b_ref):
    x = x_ref[...]
    a_ref[...] = jnp.dot(x, wa_ref[...], preferred_element_type=jnp.float32)
    b_ref[...] = jnp.dot(x, wb_ref[...], preferred_element_type=jnp.float32)


_ab_call = pl.pallas_call(
    _ab_body,
    grid=(_N // _BN,),
    in_specs=[
        pl.BlockSpec((_BN, _D), lambda i: (i, 0)),
        pl.BlockSpec((_D, _H), lambda i: (0, 0)),
        pl.BlockSpec((_D, _H), lambda i: (0, 0)),
    ],
    out_specs=[
        pl.BlockSpec((_BN, _H), lambda i: (i, 0)),
        pl.BlockSpec((_BN, _H), lambda i: (i, 0)),
    ],
    out_shape=[
        jax.ShapeDtypeStruct((_N, _H), jnp.float32),
        jax.ShapeDtypeStruct((_N, _H), jnp.float32),
    ],
)

_QB = 4000


def _efq_body(*refs):
    ef_refs, q_ref = refs[:8], refs[8]
    q_ref[...] = jnp.concatenate([r[...] for r in ef_refs], axis=1)


_efq_call = pl.pallas_call(
    _efq_body,
    grid=(_E // 8 // _QB,),
    in_specs=[
        pl.BlockSpec((_QB, _DE), lambda g, i=i: (i * (_E // 8 // _QB) + g, 0))
        for i in range(8)
    ],
    out_specs=pl.BlockSpec((_QB, 128), lambda g: (g, 0)),
    out_shape=jax.ShapeDtypeStruct((_E // 8, 128), jnp.float32),
)

_BC = 4000


def _tail_body(ef_ref, s_ref, w_ref, b_ref, w2_ref, b2_ref, r_ref):
    z = (
        jnp.dot(ef_ref[...], w_ref[...], preferred_element_type=jnp.float32)
        + b_ref[...]
        + s_ref[...]
    )
    h = jnp.maximum(z, z * 0.01)
    r = (
        jnp.dot(h, w2_ref[...], preferred_element_type=jnp.float32)
        + b2_ref[...]
    )
    r_ref[...] = jnp.maximum(r, r * 0.01)


_tail_call = pl.pallas_call(
    _tail_body,
    grid=(_E // 8 // _BC,),
    in_specs=[
        pl.BlockSpec((_BC, 128), lambda i: (i, 0)),
        pl.BlockSpec((_BC, 128), lambda i: (i, 0)),
        pl.BlockSpec((128, 128), lambda i: (0, 0)),
        pl.BlockSpec((1, 128), lambda i: (0, 0)),
        pl.BlockSpec((128, 8), lambda i: (0, 0)),
        pl.BlockSpec((1, 8), lambda i: (0, 0)),
    ],
    out_specs=pl.BlockSpec((_BC, 8), lambda i: (i, 0)),
    out_shape=jax.ShapeDtypeStruct((_E // 8, 8), jnp.float32),
)

_mesh = plsc.VectorSubcoreMesh(
    core_axis_name="core", subcore_axis_name="subcore",
    num_cores=_NC, num_subcores=_NS,
)


_sc_params = pltpu.CompilerParams(
    needs_layout_passes=False, use_tc_tiling_on_sc=False)


@functools.partial(
    pl.kernel,
    out_type=jax.ShapeDtypeStruct((_E, _H), jnp.float32),
    mesh=_mesh,
    compiler_params=_sc_params,
    scratch_types=[
        pltpu.VMEM((_CH,), jnp.int32),
        pltpu.VMEM((_CH,), jnp.int32),
        pltpu.VMEM((_SUB, _H), jnp.float32),
        pltpu.VMEM((_SUB, _H), jnp.float32),
        pltpu.VMEM((_SUB, _H), jnp.float32),
        pltpu.VMEM((_SUB, _H), jnp.float32),
        pltpu.SemaphoreType.DMA,
        pltpu.SemaphoreType.DMA,
        pltpu.SemaphoreType.DMA,
        pltpu.SemaphoreType.DMA,
        pltpu.SemaphoreType.DMA,
        pltpu.SemaphoreType.DMA,
    ],
)
def _sc_gather(a_hbm, b_hbm, ei_hbm, s_hbm, io_v, id_v, g0_v, g1_v, g2_v,
               g3_v, sem_a0, sem_a1, sem_b0, sem_b1, sem_o0, sem_o1):
    wid = lax.axis_index("subcore") * _NC + lax.axis_index("core")
    gbufs = [g0_v, g1_v, g2_v, g3_v]
    sems_a = [sem_a0, sem_a1]
    sems_b = [sem_b0, sem_b1]
    sems_o = [sem_o0, sem_o1]

    def chunk_body(t, carry):
        ci = jnp.minimum(t * _NW + wid, _NCHUNK - 1)
        cbase = pl.multiple_of(ci * _CH, _CH)
        pltpu.sync_copy(ei_hbm.at[pl.ds(cbase, _CH)], io_v)
        pltpu.sync_copy(ei_hbm.at[pl.ds(_E + cbase, _CH)], id_v)

        desc_a, desc_b, desc_o = {}, {}, {}

        def fire_a(s):
            if 0 <= s < _NSUB:
                if s - 4 >= 0:
                    desc_o.pop(s - 4).wait()
                desc_a[s] = pltpu.async_copy(
                    a_hbm.at[io_v.at[pl.ds(s * _SUB, _SUB)]],
                    gbufs[s % 4], sems_a[s % 2])

        def advance_b(s):
            if 0 <= s < _NSUB:
                desc_a.pop(s).wait()
                desc_b[s] = pltpu.async_copy(
                    b_hbm.at[id_v.at[pl.ds(s * _SUB, _SUB)]],
                    gbufs[s % 4], sems_b[s % 2], add=True)

        def advance_o(s):
            if 0 <= s < _NSUB:
                desc_b.pop(s).wait()
                desc_o[s] = pltpu.async_copy(
                    gbufs[s % 4],
                    s_hbm.at[pl.ds(cbase + s * _SUB, _SUB)], sems_o[s % 2])

        fire_a(0)
        fire_a(1)
        for s in range(_NSUB):
            advance_b(s)
            fire_a(s + 2)
            advance_o(s - 1)
        advance_o(_NSUB - 1)
        for s in sorted(desc_o):
            desc_o.pop(s).wait()

        return carry

    lax.fori_loop(0, _TMAX, chunk_body, 0)


def kernel(x, edge_index, edge_features, W1, b1, W2, b2):
    wa = W1[:_D]
    wb = W1[_D:2 * _D]
    we = W1[2 * _D:]
    a, b = _ab_call(x, wa, wb)
    ei_planar = edge_index.reshape(2, 8, _E // 8).transpose(0, 2, 1)
    ef_q = _efq_call(*([edge_features] * 8))
    we_blk = jnp.kron(jnp.eye(8, dtype=jnp.float32), we)
    w2_blk = jnp.kron(jnp.eye(8, dtype=jnp.float32), W2)
    s = _sc_gather(a, b, ei_planar.reshape(2 * _E))
    res8 = _tail_call(ef_q, s.reshape(_E // 8, 128), we_blk,
                      jnp.tile(b1, 8).reshape(1, 128), w2_blk,
                      jnp.broadcast_to(b2, (1, 8)))
    return res8.T.reshape(_E)


# speedup vs baseline: 1.3474x; 1.3474x over previous
import functools

import jax
import jax.numpy as jnp
from jax import lax
from jax.experimental import pallas as pl
from jax.experimental.pallas import tpu as pltpu
from jax.experimental.pallas import tpu_sc as plsc

_N, _E, _D, _DE, _H = 10000, 320000, 128, 16, 16

_NC, _NS = 2, 16
_NW = _NC * _NS
_CH = 2560
_NCHUNK = _E // _CH
_TMAX = -(-_NCHUNK // _NW)
_SUB = 128
_NSUB = _CH // _SUB
_NRING = 8


_BN = 2000


def _ab_body(x_ref, wa_ref, wb_ref, a_ref, b_ref):
    x = x_ref[...]
    a_ref[...] = jnp.dot(x, wa_ref[...], preferred_element_type=jnp.float32)
    b_ref[...] = jnp.dot(x, wb_ref[...], preferred_element_type=jnp.float32)


_ab_call = pl.pallas_call(
    _ab_body,
    grid=(_N // _BN,),
    in_specs=[
        pl.BlockSpec((_BN, _D), lambda i: (i, 0)),
        pl.BlockSpec((_D, _H), lambda i: (0, 0)),
        pl.BlockSpec((_D, _H), lambda i: (0, 0)),
    ],
    out_specs=[
        pl.BlockSpec((_BN, _H), lambda i: (i, 0)),
        pl.BlockSpec((_BN, _H), lambda i: (i, 0)),
    ],
    out_shape=[
        jax.ShapeDtypeStruct((_N, _H), jnp.float32),
        jax.ShapeDtypeStruct((_N, _H), jnp.float32),
    ],
)


_BC = 4000


def _tail_body(ef_ref, s_ref, w_ref, b_ref, w2_ref, b2_ref, r_ref):
    z = (
        jnp.dot(ef_ref[...], w_ref[...], preferred_element_type=jnp.float32)
        + b_ref[...]
        + s_ref[...]
    )
    h = jnp.maximum(z, z * 0.01)
    r = (
        jnp.dot(h, w2_ref[...], preferred_element_type=jnp.float32)
        + b2_ref[...]
    )
    r_ref[...] = jnp.maximum(r, r * 0.01)


_tail_call = pl.pallas_call(
    _tail_body,
    grid=(_E // 8 // _BC,),
    in_specs=[
        pl.BlockSpec((_BC, 128), lambda i: (i, 0)),
        pl.BlockSpec((_BC, 128), lambda i: (i, 0)),
        pl.BlockSpec((128, 128), lambda i: (0, 0)),
        pl.BlockSpec((1, 128), lambda i: (0, 0)),
        pl.BlockSpec((128, 8), lambda i: (0, 0)),
        pl.BlockSpec((1, 8), lambda i: (0, 0)),
    ],
    out_specs=pl.BlockSpec((_BC, 8), lambda i: (i, 0)),
    out_shape=jax.ShapeDtypeStruct((_E // 8, 8), jnp.float32),
)

_mesh = plsc.VectorSubcoreMesh(
    core_axis_name="core", subcore_axis_name="subcore",
    num_cores=_NC, num_subcores=_NS,
)


_sc_params = pltpu.CompilerParams(
    needs_layout_passes=False, use_tc_tiling_on_sc=False)


@functools.partial(
    pl.kernel,
    out_type=jax.ShapeDtypeStruct((_E, _H), jnp.float32),
    mesh=_mesh,
    compiler_params=_sc_params,
    scratch_types=(
        [
            pltpu.VMEM((_CH,), jnp.int32),
            pltpu.VMEM((_CH,), jnp.int32),
        ]
        + [pltpu.VMEM((_SUB, _H), jnp.float32)] * _NRING
        + [pltpu.SemaphoreType.DMA] * 12
    ),
)
def _sc_gather(a_hbm, b_hbm, ei_hbm, s_hbm, io_v, id_v, *rest):
    gbufs = list(rest[:_NRING])
    sems_a = list(rest[_NRING:_NRING + 4])
    sems_b = list(rest[_NRING + 4:_NRING + 8])
    sems_o = list(rest[_NRING + 8:_NRING + 12])
    wid = lax.axis_index("subcore") * _NC + lax.axis_index("core")

    def chunk_body(t, carry):
        ci = jnp.minimum(t * _NW + wid, _NCHUNK - 1)
        cbase = pl.multiple_of(ci * _CH, _CH)
        pltpu.sync_copy(ei_hbm.at[pl.ds(cbase, _CH)], io_v)
        pltpu.sync_copy(ei_hbm.at[pl.ds(_E + cbase, _CH)], id_v)

        desc_a, desc_b, desc_o = {}, {}, {}

        def fire_a(s):
            if 0 <= s < _NSUB:
                if s - _NRING >= 0:
                    desc_o.pop(s - _NRING).wait()
                desc_a[s] = pltpu.async_copy(
                    a_hbm.at[io_v.at[pl.ds(s * _SUB, _SUB)]],
                    gbufs[s % _NRING], sems_a[s % 4])

        def advance_b(s):
            if 0 <= s < _NSUB:
                desc_a.pop(s).wait()
                desc_b[s] = pltpu.async_copy(
                    b_hbm.at[id_v.at[pl.ds(s * _SUB, _SUB)]],
                    gbufs[s % _NRING], sems_b[s % 4], add=True)

        def advance_o(s):
            if 0 <= s < _NSUB:
                desc_b.pop(s).wait()
                desc_o[s] = pltpu.async_copy(
                    gbufs[s % _NRING],
                    s_hbm.at[pl.ds(cbase + s * _SUB, _SUB)], sems_o[s % 4])

        for s in range(4):
            fire_a(s)
        for s in range(_NSUB):
            advance_b(s)
            fire_a(s + 4)
            advance_o(s - 2)
        advance_o(_NSUB - 2)
        advance_o(_NSUB - 1)
        for s in sorted(desc_o):
            desc_o.pop(s).wait()

        return carry

    lax.fori_loop(0, _TMAX, chunk_body, 0)


def kernel(x, edge_index, edge_features, W1, b1, W2, b2):
    wa = W1[:_D]
    wb = W1[_D:2 * _D]
    we = W1[2 * _D:]
    a, b = _ab_call(x, wa, wb)
    ef_q = edge_features.reshape(_E // 8, 8 * _DE)
    we_blk = jnp.kron(jnp.eye(8, dtype=jnp.float32), we)
    w2_blk = jnp.kron(jnp.eye(8, dtype=jnp.float32), W2)
    s = _sc_gather(a, b, edge_index.reshape(2 * _E))
    res8 = _tail_call(ef_q, s.reshape(_E // 8, 128), we_blk,
                      jnp.tile(b1, 8).reshape(1, 128), w2_blk,
                      jnp.broadcast_to(b2, (1, 8)))
    return res8.reshape(_E)


# speedup vs baseline: 1.3656x; 1.0136x over previous
import functools

import jax
import jax.numpy as jnp
from jax import lax
from jax.experimental import pallas as pl
from jax.experimental.pallas import tpu as pltpu
from jax.experimental.pallas import tpu_sc as plsc

_N, _E, _D, _DE, _H = 10000, 320000, 128, 16, 16

_NC, _NS = 2, 16
_NW = _NC * _NS
_CH = 1280
_NCHUNK = _E // _CH
_TMAX = -(-_NCHUNK // _NW)
_SUB = 128
_NSUB = _CH // _SUB
_NRING = 8


_BN = 2000


def _ab_body(x_ref, wa_ref, wb_ref, a_ref, b_ref):
    x = x_ref[...]
    a_ref[...] = jnp.dot(x, wa_ref[...], preferred_element_type=jnp.float32)
    b_ref[...] = jnp.dot(x, wb_ref[...], preferred_element_type=jnp.float32)


_ab_call = pl.pallas_call(
    _ab_body,
    grid=(_N // _BN,),
    in_specs=[
        pl.BlockSpec((_BN, _D), lambda i: (i, 0)),
        pl.BlockSpec((_D, _H), lambda i: (0, 0)),
        pl.BlockSpec((_D, _H), lambda i: (0, 0)),
    ],
    out_specs=[
        pl.BlockSpec((_BN, _H), lambda i: (i, 0)),
        pl.BlockSpec((_BN, _H), lambda i: (i, 0)),
    ],
    out_shape=[
        jax.ShapeDtypeStruct((_N, _H), jnp.float32),
        jax.ShapeDtypeStruct((_N, _H), jnp.float32),
    ],
)


_BC = 4000


def _tail_body(ef_ref, s_ref, w_ref, b_ref, w2_ref, b2_ref, r_ref):
    z = (
        jnp.dot(ef_ref[...], w_ref[...], preferred_element_type=jnp.float32)
        + b_ref[...]
        + s_ref[...]
    )
    h = jnp.maximum(z, z * 0.01)
    r = (
        jnp.dot(h, w2_ref[...], preferred_element_type=jnp.float32)
        + b2_ref[...]
    )
    r_ref[...] = jnp.maximum(r, r * 0.01)


_tail_call = pl.pallas_call(
    _tail_body,
    grid=(_E // 8 // _BC,),
    in_specs=[
        pl.BlockSpec((_BC, 128), lambda i: (i, 0)),
        pl.BlockSpec((_BC, 128), lambda i: (i, 0)),
        pl.BlockSpec((128, 128), lambda i: (0, 0)),
        pl.BlockSpec((1, 128), lambda i: (0, 0)),
        pl.BlockSpec((128, 8), lambda i: (0, 0)),
        pl.BlockSpec((1, 8), lambda i: (0, 0)),
    ],
    out_specs=pl.BlockSpec((_BC, 8), lambda i: (i, 0)),
    out_shape=jax.ShapeDtypeStruct((_E // 8, 8), jnp.float32),
)

_mesh = plsc.VectorSubcoreMesh(
    core_axis_name="core", subcore_axis_name="subcore",
    num_cores=_NC, num_subcores=_NS,
)


def _ei_body(ei_ref, og_ref, dg_ref):
    z = ei_ref[...]
    og_ref[...] = z[0, :]
    dg_ref[...] = z[1, :]


_ei_call = pl.pallas_call(
    _ei_body,
    out_shape=[
        jax.ShapeDtypeStruct((_E,), jnp.int32),
        jax.ShapeDtypeStruct((_E,), jnp.int32),
    ],
)

_sc_params = pltpu.CompilerParams(
    needs_layout_passes=False, use_tc_tiling_on_sc=False)


@functools.partial(
    pl.kernel,
    out_type=jax.ShapeDtypeStruct((_E, _H), jnp.float32),
    mesh=_mesh,
    compiler_params=_sc_params,
    scratch_types=(
        [
            pltpu.VMEM((_CH,), jnp.int32),
            pltpu.VMEM((_CH,), jnp.int32),
        ]
        + [pltpu.VMEM((_SUB, _H), jnp.float32)] * _NRING
        + [pltpu.SemaphoreType.DMA] * 12
    ),
)
def _sc_gather(a_hbm, b_hbm, og_hbm, dg_hbm, s_hbm, io_v, id_v, *rest):
    gbufs = list(rest[:_NRING])
    sems_a = list(rest[_NRING:_NRING + 4])
    sems_b = list(rest[_NRING + 4:_NRING + 8])
    sems_o = list(rest[_NRING + 8:_NRING + 12])
    wid = lax.axis_index("subcore") * _NC + lax.axis_index("core")

    def chunk_body(t, carry):
        ci = jnp.minimum(t * _NW + wid, _NCHUNK - 1)
        cbase = pl.multiple_of(ci * _CH, _CH)
        pltpu.sync_copy(og_hbm.at[pl.ds(cbase, _CH)], io_v)
        pltpu.sync_copy(dg_hbm.at[pl.ds(cbase, _CH)], id_v)

        desc_a, desc_b, desc_o = {}, {}, {}

        def fire_a(s):
            if 0 <= s < _NSUB:
                if s - _NRING >= 0:
                    desc_o.pop(s - _NRING).wait()
                desc_a[s] = pltpu.async_copy(
                    a_hbm.at[io_v.at[pl.ds(s * _SUB, _SUB)]],
                    gbufs[s % _NRING], sems_a[s % 4])

        def advance_b(s):
            if 0 <= s < _NSUB:
                desc_a.pop(s).wait()
                desc_b[s] = pltpu.async_copy(
                    b_hbm.at[id_v.at[pl.ds(s * _SUB, _SUB)]],
                    gbufs[s % _NRING], sems_b[s % 4], add=True)

        def advance_o(s):
            if 0 <= s < _NSUB:
                desc_b.pop(s).wait()
                desc_o[s] = pltpu.async_copy(
                    gbufs[s % _NRING],
                    s_hbm.at[pl.ds(cbase + s * _SUB, _SUB)], sems_o[s % 4])

        for s in range(4):
            fire_a(s)
        for s in range(_NSUB):
            advance_b(s)
            fire_a(s + 4)
            advance_o(s - 2)
        advance_o(_NSUB - 2)
        advance_o(_NSUB - 1)
        for s in sorted(desc_o):
            desc_o.pop(s).wait()

        return carry

    lax.fori_loop(0, _TMAX, chunk_body, 0)


def kernel(x, edge_index, edge_features, W1, b1, W2, b2):
    wa = W1[:_D]
    wb = W1[_D:2 * _D]
    we = W1[2 * _D:]
    a, b = _ab_call(x, wa, wb)
    ef_q = edge_features.reshape(_E // 8, 8 * _DE)
    we_blk = jnp.kron(jnp.eye(8, dtype=jnp.float32), we)
    w2_blk = jnp.kron(jnp.eye(8, dtype=jnp.float32), W2)
    og, dg = _ei_call(edge_index)
    s = _sc_gather(a, b, og, dg)
    res8 = _tail_call(ef_q, s.reshape(_E // 8, 128), we_blk,
                      jnp.tile(b1, 8).reshape(1, 128), w2_blk,
                      jnp.broadcast_to(b2, (1, 8)))
    return res8.reshape(_E)
